# Initial kernel scaffold; baseline (speedup 1.0000x reference)
import jax
import jax.numpy as jnp
from jax.experimental import pallas as pl


def kernel(x, edge_index, edge_attr, batch, eps, W1, b1, g1, bt1, W2, b2, bond_emb, g_out, bt_out):
    raise NotImplementedError("write your pallas kernel here")


# speedup vs baseline: 4.8975x; 4.8975x over previous
import functools

import jax
import jax.numpy as jnp
from jax import lax
from jax.experimental import pallas as pl
from jax.experimental.pallas import tpu as pltpu
from jax.experimental.pallas import tpu_sc as plsc

N = 10000
D = 128
L = 3
NPAD = 10240
EPG = 128
NC = 2
NS = 16
NW = NC * NS
E = 320000
G = -(-E // (NW * EPG))
EPW = G * EPG
EPAD = NW * EPW
RPT = NPAD // NS


def _mp_body(h_hbm, src_hbm, dst_hbm, cidx_hbm, ctab_hbm, out_hbm,
             src_v, dst_v, cidx_v, hrows, erows, agg, sem1, sem2):
    cid = lax.axis_index("c")
    sid = lax.axis_index("s")
    wid = sid * NC + cid

    def zbody(r, _):
        for c in range(D // 16):
            erows[r, pl.ds(c * 16, 16)] = jnp.zeros((16,), jnp.float32)
        return 0
    lax.fori_loop(0, EPG, zbody, 0)
    for k in range(RPT // EPG):
        pltpu.sync_copy(erows, agg.at[pl.ds(sid * RPT + k * EPG, EPG)])
    plsc.subcore_barrier()

    def gbody(g, _):
        off = wid * EPW + g * EPG
        pltpu.sync_copy(src_hbm.at[pl.ds(off, EPG)], src_v)
        pltpu.sync_copy(cidx_hbm.at[pl.ds(off, EPG)], cidx_v)
        pltpu.sync_copy(dst_hbm.at[pl.ds(off, EPG)], dst_v)
        cp1 = pltpu.async_copy(h_hbm.at[src_v], hrows, sem1)
        cp2 = pltpu.async_copy(ctab_hbm.at[cidx_v], erows, sem2)
        cp1.wait()
        cp2.wait()

        def cbody(r, _):
            for c in range(D // 16):
                s = pl.ds(c * 16, 16)
                hrows[r, s] = jnp.maximum(hrows[r, s] + erows[r, s], 0.0)
            return 0
        lax.fori_loop(0, EPG, cbody, 0)
        pltpu.sync_copy(hrows, agg.at[dst_v], add=True)
        return 0
    lax.fori_loop(0, G, gbody, 0)

    plsc.subcore_barrier()
    pltpu.sync_copy(agg.at[pl.ds(sid * RPT, RPT)],
                    out_hbm.at[cid, pl.ds(sid * RPT, RPT)])


def _mp_call(h, srcp, dstp, cidxp, ctab_l):
    mesh = plsc.VectorSubcoreMesh(core_axis_name="c", subcore_axis_name="s")
    f = pl.kernel(
        _mp_body,
        out_type=jax.ShapeDtypeStruct((NC, NPAD, D), jnp.float32),
        mesh=mesh,
        scratch_types=[
            pltpu.VMEM((EPG,), jnp.int32),
            pltpu.VMEM((EPG,), jnp.int32),
            pltpu.VMEM((EPG,), jnp.int32),
            pltpu.VMEM((EPG, D), jnp.float32),
            pltpu.VMEM((EPG, D), jnp.float32),
            pltpu.VMEM_SHARED((NPAD, D), jnp.float32),
            pltpu.SemaphoreType.DMA,
            pltpu.SemaphoreType.DMA,
        ],
    )
    return f(h, srcp, dstp, cidxp, ctab_l)


def _mlp_body(relu_out, h_ref, a_ref, w1_ref, b1_ref, g1_ref, t1_ref,
              w2_ref, b2_ref, go_ref, to_ref, eps_ref, out_ref):
    h = h_ref[...]
    agg = a_ref[0, 0:N, :] + a_ref[1, 0:N, :]
    z0 = (1.0 + eps_ref[0, 0]) * h + agg
    z1 = jnp.dot(z0, w1_ref[...], preferred_element_type=jnp.float32) + b1_ref[...]
    mu = jnp.mean(z1, axis=0, keepdims=True)
    var = jnp.mean((z1 - mu) ** 2, axis=0, keepdims=True)
    z1 = (z1 - mu) / jnp.sqrt(var + 1e-5) * g1_ref[...] + t1_ref[...]
    z1 = jnp.maximum(z1, 0.0)
    z2 = jnp.dot(z1, w2_ref[...], preferred_element_type=jnp.float32) + b2_ref[...]
    mu2 = jnp.mean(z2, axis=0, keepdims=True)
    var2 = jnp.mean((z2 - mu2) ** 2, axis=0, keepdims=True)
    z2 = (z2 - mu2) / jnp.sqrt(var2 + 1e-5) * go_ref[...] + to_ref[...]
    if relu_out:
        z2 = jnp.maximum(z2, 0.0)
    out_ref[...] = z2 + h


def _mlp_call(h, parts, w1, b1v, g1v, t1v, w2, b2v, gov, tov, eps_l, relu_out):
    body = functools.partial(_mlp_body, relu_out)
    vspec = pl.BlockSpec(memory_space=pltpu.VMEM)
    return pl.pallas_call(
        body,
        out_shape=jax.ShapeDtypeStruct((N, D), jnp.float32),
        in_specs=[vspec] * 10 + [pl.BlockSpec(memory_space=pltpu.SMEM)],
        out_specs=vspec,
    )(h, parts, w1, b1v, g1v, t1v, w2, b2v, gov, tov, eps_l)


def kernel(x, edge_index, edge_attr, batch, eps, W1, b1, g1, bt1, W2, b2, bond_emb, g_out, bt_out):
    src = edge_index[0]
    dst = edge_index[1]
    cidx = edge_attr[:, 0] * 25 + edge_attr[:, 1] * 5 + edge_attr[:, 2]
    srcp = jnp.pad(src, (0, EPAD - E))
    dstp = jnp.pad(dst, (0, EPAD - E), constant_values=N)
    cidxp = jnp.pad(cidx, (0, EPAD - E))
    ctab = (bond_emb[:, 0][:, :, None, None, :]
            + bond_emb[:, 1][:, None, :, None, :]
            + bond_emb[:, 2][:, None, None, :, :]).reshape(L, 125, D)
    ctab = jnp.pad(ctab, ((0, 0), (0, 3), (0, 0)))

    h = x
    for l in range(L):
        parts = _mp_call(h, srcp, dstp, cidxp, ctab[l])
        h = _mlp_call(h, parts,
                      W1[l], b1[l][None], g1[l][None], bt1[l][None],
                      W2[l], b2[l][None], g_out[l][None], bt_out[l][None],
                      eps[l].reshape(1, 1), relu_out=(l < L - 1))
    return h
